# baseline (speedup 1.0000x reference)
import functools

import jax
import jax.numpy as jnp
from jax import lax
from jax.experimental import pallas as pl
from jax.experimental.pallas import tpu as pltpu
from jax.experimental.pallas import tpu_sc as plsc

_LANES = 16


def _make_sc_gather(B, V, NC, NS):
    NW = NC * NS
    bpw = B // NW

    mesh = plsc.VectorSubcoreMesh(core_axis_name="c", subcore_axis_name="s")

    @functools.partial(
        pl.kernel,
        mesh=mesh,
        out_type=jax.ShapeDtypeStruct((B,), jnp.float32),
        compiler_params=pltpu.CompilerParams(needs_layout_passes=False),
        scratch_types=[
            pltpu.VMEM((bpw, 2), jnp.int32),
            pltpu.VMEM((bpw,), jnp.int32),
            pltpu.VMEM((bpw,), jnp.float32),
            pltpu.SemaphoreType.DMA,
        ],
    )
    def run(ids_hbm, bias_hbm, out_hbm, ids_v, uid_v, out_v, sem):
        wid = lax.axis_index("s") * NC + lax.axis_index("c")
        base = wid * bpw
        pltpu.sync_copy(ids_hbm.at[pl.ds(base, bpw), :], ids_v)

        lane = lax.iota(jnp.int32, _LANES)
        col0 = jnp.zeros((_LANES,), jnp.int32)

        def body(j, carry):
            ridx = j * _LANES + lane
            uid = plsc.load_gather(ids_v, [ridx, col0])
            plsc.store_scatter(uid_v, [ridx], uid)
            return carry

        lax.fori_loop(0, bpw // _LANES, body, 0)

        pltpu.async_copy(bias_hbm.at[uid_v], out_v, sem).wait()
        pltpu.sync_copy(out_v, out_hbm.at[pl.ds(base, bpw)])

    return run


def kernel(x, user_ids, bias):
    B = x.shape[0]
    V = bias.shape[0]
    info = plsc.get_sparse_core_info()
    NC, NS = info.num_cores, info.num_subcores

    run = _make_sc_gather(B, V, NC, NS)
    gathered = run(user_ids, bias)
    return x + gathered[:, None]


# speedup vs baseline: 1.2719x; 1.2719x over previous
import functools

import jax
import jax.numpy as jnp
from jax import lax
from jax.experimental import pallas as pl
from jax.experimental.pallas import tpu as pltpu
from jax.experimental.pallas import tpu_sc as plsc

_CHUNKS = 4


def _make_sc_gather(B, V, NC, NS):
    NW = NC * NS
    bpw = B // NW
    cs = bpw // _CHUNKS

    mesh = plsc.VectorSubcoreMesh(core_axis_name="c", subcore_axis_name="s")

    @functools.partial(
        pl.kernel,
        mesh=mesh,
        out_type=jax.ShapeDtypeStruct((B,), jnp.float32),
        compiler_params=pltpu.CompilerParams(needs_layout_passes=False),
        scratch_types=[
            pltpu.VMEM((bpw,), jnp.int32),
            pltpu.VMEM((bpw,), jnp.float32),
            pltpu.SemaphoreType.DMA((_CHUNKS,)),
            pltpu.SemaphoreType.DMA((_CHUNKS,)),
            pltpu.SemaphoreType.DMA((_CHUNKS,)),
        ],
    )
    def run(uid_hbm, bias_hbm, out_hbm, uid_v, out_v, sem_in, sem_g, sem_out):
        wid = lax.axis_index("s") * NC + lax.axis_index("c")
        base = wid * bpw

        in_cp = [
            pltpu.async_copy(
                uid_hbm.at[pl.ds(base + c * cs, cs)],
                uid_v.at[pl.ds(c * cs, cs)],
                sem_in.at[c],
            )
            for c in range(_CHUNKS)
        ]
        g_cp = []
        for c in range(_CHUNKS):
            in_cp[c].wait()
            g_cp.append(
                pltpu.async_copy(
                    bias_hbm.at[uid_v.at[pl.ds(c * cs, cs)]],
                    out_v.at[pl.ds(c * cs, cs)],
                    sem_g.at[c],
                )
            )
        out_cp = []
        for c in range(_CHUNKS):
            g_cp[c].wait()
            out_cp.append(
                pltpu.async_copy(
                    out_v.at[pl.ds(c * cs, cs)],
                    out_hbm.at[pl.ds(base + c * cs, cs)],
                    sem_out.at[c],
                )
            )
        for c in range(_CHUNKS):
            out_cp[c].wait()

    return run


def kernel(x, user_ids, bias):
    B = x.shape[0]
    V = bias.shape[0]
    info = plsc.get_sparse_core_info()
    NC, NS = info.num_cores, info.num_subcores

    run = _make_sc_gather(B, V, NC, NS)
    gathered = run(user_ids[:, 0], bias)
    return x + gathered[:, None]


# speedup vs baseline: 1.2847x; 1.0101x over previous
import functools

import jax
import jax.numpy as jnp
from jax import lax
from jax.experimental import pallas as pl
from jax.experimental.pallas import tpu as pltpu
from jax.experimental.pallas import tpu_sc as plsc

_CHUNKS = 2


def _make_sc_gather(B, V, NC, NS):
    NW = NC * NS
    bpw = B // NW
    cs = bpw // _CHUNKS

    mesh = plsc.VectorSubcoreMesh(core_axis_name="c", subcore_axis_name="s")

    @functools.partial(
        pl.kernel,
        mesh=mesh,
        out_type=jax.ShapeDtypeStruct((B,), jnp.float32),
        compiler_params=pltpu.CompilerParams(needs_layout_passes=False),
        scratch_types=[
            pltpu.VMEM((bpw,), jnp.int32),
            pltpu.VMEM((bpw,), jnp.float32),
            pltpu.SemaphoreType.DMA((_CHUNKS,)),
            pltpu.SemaphoreType.DMA((_CHUNKS,)),
            pltpu.SemaphoreType.DMA((_CHUNKS,)),
        ],
    )
    def run(uid_hbm, bias_hbm, out_hbm, uid_v, out_v, sem_in, sem_g, sem_out):
        wid = lax.axis_index("s") * NC + lax.axis_index("c")
        base = wid * bpw

        in_cp = [
            pltpu.async_copy(
                uid_hbm.at[pl.ds(base + c * cs, cs)],
                uid_v.at[pl.ds(c * cs, cs)],
                sem_in.at[c],
            )
            for c in range(_CHUNKS)
        ]
        g_cp = []
        for c in range(_CHUNKS):
            in_cp[c].wait()
            g_cp.append(
                pltpu.async_copy(
                    bias_hbm.at[uid_v.at[pl.ds(c * cs, cs)]],
                    out_v.at[pl.ds(c * cs, cs)],
                    sem_g.at[c],
                )
            )
        out_cp = []
        for c in range(_CHUNKS):
            g_cp[c].wait()
            out_cp.append(
                pltpu.async_copy(
                    out_v.at[pl.ds(c * cs, cs)],
                    out_hbm.at[pl.ds(base + c * cs, cs)],
                    sem_out.at[c],
                )
            )
        for c in range(_CHUNKS):
            out_cp[c].wait()

    return run


def kernel(x, user_ids, bias):
    B = x.shape[0]
    V = bias.shape[0]
    info = plsc.get_sparse_core_info()
    NC, NS = info.num_cores, info.num_subcores

    run = _make_sc_gather(B, V, NC, NS)
    gathered = run(user_ids[:, 0], bias)
    return x + gathered[:, None]


# speedup vs baseline: 1.2913x; 1.0052x over previous
import functools

import jax
import jax.numpy as jnp
from jax import lax
from jax.experimental import pallas as pl
from jax.experimental.pallas import tpu as pltpu
from jax.experimental.pallas import tpu_sc as plsc


def _make_sc_gather(B, V, NC, NS):
    NW = NC * NS
    bpw = B // NW

    mesh = plsc.VectorSubcoreMesh(core_axis_name="c", subcore_axis_name="s")

    @functools.partial(
        pl.kernel,
        mesh=mesh,
        out_type=jax.ShapeDtypeStruct((B,), jnp.float32),
        compiler_params=pltpu.CompilerParams(needs_layout_passes=False),
        scratch_types=[
            pltpu.VMEM((bpw,), jnp.int32),
            pltpu.VMEM((bpw,), jnp.float32),
            pltpu.SemaphoreType.DMA,
        ],
    )
    def run(uid_hbm, bias_hbm, out_hbm, uid_v, out_v, sem):
        wid = lax.axis_index("s") * NC + lax.axis_index("c")
        base = wid * bpw
        pltpu.sync_copy(uid_hbm.at[pl.ds(base, bpw)], uid_v)
        pltpu.async_copy(bias_hbm.at[uid_v], out_v, sem).wait()
        pltpu.sync_copy(out_v, out_hbm.at[pl.ds(base, bpw)])

    return run


def kernel(x, user_ids, bias):
    B = x.shape[0]
    V = bias.shape[0]
    info = plsc.get_sparse_core_info()
    NC, NS = info.num_cores, info.num_subcores

    run = _make_sc_gather(B, V, NC, NS)
    gathered = run(user_ids[:, 0], bias)
    return x + gathered[:, None]
